# initial kernel scaffold (unmeasured)
import jax
import jax.numpy as jnp
from jax import lax
from jax.experimental import pallas as pl
from jax.experimental.pallas import tpu as pltpu

N_DEV = 16
HL = 4
DH = 64
QB = 4
F32 = jnp.float32


def kernel(x, Wq, K_ext, V_ext, Wo):
    B, Sq, Dm = x.shape
    Skv_l = K_ext.shape[1]

    def body(x_ref, wq_ref, k_ref, v_ref, wo_ref, out_ref,
             q_ref, kall_ref, vall_ref, ctx_ref, ar_ref,
             ksend_sems, vsend_sems, krecv_sems, vrecv_sems,
             ar_send_sems, ar_recv_sems):
        my = lax.axis_index("i")

        def a2a_desc(d, tensor):
            src, dst, ssem, rsem = (
                (k_ref, kall_ref, ksend_sems, krecv_sems)
                if tensor == "k"
                else (v_ref, vall_ref, vsend_sems, vrecv_sems)
            )
            return pltpu.make_async_remote_copy(
                src_ref=src.at[:, :, pl.ds(4 * d, 4), :],
                dst_ref=dst.at[my],
                send_sem=ssem.at[d],
                recv_sem=rsem.at[my],
                device_id=(d,),
                device_id_type=pl.DeviceIdType.MESH,
            )

        def recv_desc(j, tensor):
            dst, ssem, rsem = (
                (kall_ref, ksend_sems, krecv_sems)
                if tensor == "k"
                else (vall_ref, vsend_sems, vrecv_sems)
            )
            return pltpu.make_async_remote_copy(
                src_ref=dst.at[j],
                dst_ref=dst.at[j],
                send_sem=ssem.at[j],
                recv_sem=rsem.at[j],
                device_id=(j,),
                device_id_type=pl.DeviceIdType.MESH,
            )

        barrier = pltpu.get_barrier_semaphore()
        for p in range(N_DEV):
            @pl.when(my != p)
            def _(p=p):
                pl.semaphore_signal(
                    barrier, inc=1,
                    device_id=(p,), device_id_type=pl.DeviceIdType.MESH,
                )
        pl.semaphore_wait(barrier, N_DEV - 1)

        for d in range(N_DEV):
            @pl.when(my == d)
            def _(d=d):
                kall_ref[d] = k_ref[:, :, 4 * d:4 * (d + 1), :]
                vall_ref[d] = v_ref[:, :, 4 * d:4 * (d + 1), :]

            @pl.when(my != d)
            def _(d=d):
                a2a_desc(d, "k").start()
                a2a_desc(d, "v").start()

        for b in range(B):
            q_ref[b] = jnp.dot(
                x_ref[b], wq_ref[...], preferred_element_type=F32
            )

        for j in range(N_DEV):
            @pl.when(my != j)
            def _(j=j):
                recv_desc(j, "k").wait_recv()
                recv_desc(j, "v").wait_recv()

        for b in range(B):
            for qb in range(QB):
                for h in range(HL):
                    q2 = q_ref[b, pl.ds(qb * 64, 64), pl.ds(h * 64, 64)]
                    k2 = kall_ref[:, b, pl.ds(qb * 64, 64), h, :]
                    k2 = k2.reshape(N_DEV * 64, DH)
                    s = lax.dot_general(
                        q2, k2, (((1,), (1,)), ((), ())),
                        preferred_element_type=F32,
                    ) * 0.125
                    m = jnp.max(s, axis=1, keepdims=True)
                    w = jnp.exp(s - m)
                    p_ = w / jnp.sum(w, axis=1, keepdims=True)
                    v2 = vall_ref[:, b, pl.ds(qb * 64, 64), h, :]
                    v2 = v2.reshape(N_DEV * 64, DH)
                    c = jnp.dot(p_, v2, preferred_element_type=F32)
                    ctx_ref[b, pl.ds(qb * 64, 64), h, :] = c

        for b in range(B):
            acc = None
            for h in range(HL):
                t = jnp.dot(
                    ctx_ref[b, :, h, :],
                    wo_ref[pl.ds(h * 64, 64), :],
                    preferred_element_type=F32,
                )
                acc = t if acc is None else acc + t
            out_ref[b] = acc

        for d in range(N_DEV):
            @pl.when(my != d)
            def _(d=d):
                a2a_desc(d, "k").wait_send()
                a2a_desc(d, "v").wait_send()

        for s, k in enumerate([1, 2, 4, 8]):
            partner = my ^ k
            rdma = pltpu.make_async_remote_copy(
                src_ref=out_ref,
                dst_ref=ar_ref.at[s],
                send_sem=ar_send_sems.at[s],
                recv_sem=ar_recv_sems.at[s],
                device_id=(partner,),
                device_id_type=pl.DeviceIdType.MESH,
            )
            rdma.start()
            rdma.wait()
            out_ref[...] = out_ref[...] + ar_ref[s]

    return pl.pallas_call(
        body,
        out_shape=jax.ShapeDtypeStruct((B, Sq, Dm), F32),
        in_specs=[pl.BlockSpec(memory_space=pltpu.VMEM)] * 5,
        out_specs=pl.BlockSpec(memory_space=pltpu.VMEM),
        scratch_shapes=[
            pltpu.VMEM((B, Sq, HL * DH), F32),
            pltpu.VMEM((N_DEV, B, Skv_l, HL, DH), F32),
            pltpu.VMEM((N_DEV, B, Skv_l, HL, DH), F32),
            pltpu.VMEM((B, Sq, HL, DH), F32),
            pltpu.VMEM((4, B, Sq, Dm), F32),
            pltpu.SemaphoreType.DMA((N_DEV,)),
            pltpu.SemaphoreType.DMA((N_DEV,)),
            pltpu.SemaphoreType.DMA((N_DEV,)),
            pltpu.SemaphoreType.DMA((N_DEV,)),
            pltpu.SemaphoreType.DMA((4,)),
            pltpu.SemaphoreType.DMA((4,)),
        ],
        compiler_params=pltpu.CompilerParams(collective_id=0),
    )(x, Wq, K_ext, V_ext, Wo)


# baseline (device time: 592006 ns/iter reference)
import jax
import jax.numpy as jnp
from jax import lax
from jax.experimental import pallas as pl
from jax.experimental.pallas import tpu as pltpu

N_DEV = 16
HL = 4
DH = 64
QB = 4
F32 = jnp.float32


def kernel(x, Wq, K_ext, V_ext, Wo):
    B, Sq, Dm = x.shape
    Skv_l = K_ext.shape[1]

    def body(x_ref, wq_ref, k_ref, v_ref, wo_ref, out_ref,
             q_ref, kall_ref, vall_ref, ctx_ref, ar_ref,
             ksend_sems, vsend_sems, krecv_sems, vrecv_sems,
             ar_send_sems, ar_recv_sems, local_sems):
        my = lax.axis_index("i")

        def a2a_desc(d, tensor):
            src, dst, ssem, rsem = (
                (k_ref, kall_ref, ksend_sems, krecv_sems)
                if tensor == "k"
                else (v_ref, vall_ref, vsend_sems, vrecv_sems)
            )
            return pltpu.make_async_remote_copy(
                src_ref=src.at[:, :, pl.ds(4 * d, 4), :],
                dst_ref=dst.at[my],
                send_sem=ssem.at[d],
                recv_sem=rsem.at[my],
                device_id=(d,),
                device_id_type=pl.DeviceIdType.MESH,
            )

        def recv_desc(j, tensor):
            dst, ssem, rsem = (
                (kall_ref, ksend_sems, krecv_sems)
                if tensor == "k"
                else (vall_ref, vsend_sems, vrecv_sems)
            )
            return pltpu.make_async_remote_copy(
                src_ref=dst.at[j],
                dst_ref=dst.at[j],
                send_sem=ssem.at[j],
                recv_sem=rsem.at[j],
                device_id=(j,),
                device_id_type=pl.DeviceIdType.MESH,
            )

        barrier = pltpu.get_barrier_semaphore()
        for p in range(N_DEV):
            @pl.when(my != p)
            def _(p=p):
                pl.semaphore_signal(
                    barrier, inc=1,
                    device_id=(p,), device_id_type=pl.DeviceIdType.MESH,
                )
        pl.semaphore_wait(barrier, N_DEV - 1)

        for d in range(N_DEV):
            @pl.when(my == d)
            def _(d=d):
                pltpu.make_async_copy(
                    k_ref.at[:, :, pl.ds(4 * d, 4), :],
                    kall_ref.at[d], local_sems.at[0],
                ).start()
                pltpu.make_async_copy(
                    v_ref.at[:, :, pl.ds(4 * d, 4), :],
                    vall_ref.at[d], local_sems.at[1],
                ).start()

            @pl.when(my != d)
            def _(d=d):
                a2a_desc(d, "k").start()
                a2a_desc(d, "v").start()

        for b in range(B):
            q_ref[b] = jnp.dot(
                x_ref[b], wq_ref[...], preferred_element_type=F32
            )

        for d in range(N_DEV):
            @pl.when(my == d)
            def _(d=d):
                pltpu.make_async_copy(
                    k_ref.at[:, :, pl.ds(4 * d, 4), :],
                    kall_ref.at[d], local_sems.at[0],
                ).wait()
                pltpu.make_async_copy(
                    v_ref.at[:, :, pl.ds(4 * d, 4), :],
                    vall_ref.at[d], local_sems.at[1],
                ).wait()
        for j in range(N_DEV):
            @pl.when(my != j)
            def _(j=j):
                recv_desc(j, "k").wait_recv()
                recv_desc(j, "v").wait_recv()

        for b in range(B):
            for qb in range(QB):
                for h in range(HL):
                    q2 = q_ref[b, pl.ds(qb * 64, 64), pl.ds(h * 64, 64)]
                    k2 = kall_ref[:, b, pl.ds(qb * 64, 64), h, :]
                    k2 = k2.reshape(N_DEV * 64, DH)
                    s = lax.dot_general(
                        q2, k2, (((1,), (1,)), ((), ())),
                        preferred_element_type=F32,
                    ) * 0.125
                    m = jnp.max(s, axis=1, keepdims=True)
                    w = jnp.exp(s - m)
                    p_ = w / jnp.sum(w, axis=1, keepdims=True)
                    v2 = vall_ref[:, b, pl.ds(qb * 64, 64), h, :]
                    v2 = v2.reshape(N_DEV * 64, DH)
                    c = jnp.dot(p_, v2, preferred_element_type=F32)
                    ctx_ref[b, pl.ds(qb * 64, 64), h, :] = c

        for b in range(B):
            acc = None
            for h in range(HL):
                t = jnp.dot(
                    ctx_ref[b, :, h, :],
                    wo_ref[pl.ds(h * 64, 64), :],
                    preferred_element_type=F32,
                )
                acc = t if acc is None else acc + t
            out_ref[b] = acc

        for d in range(N_DEV):
            @pl.when(my != d)
            def _(d=d):
                a2a_desc(d, "k").wait_send()
                a2a_desc(d, "v").wait_send()

        for s, k in enumerate([1, 2, 4, 8]):
            partner = my ^ k
            rdma = pltpu.make_async_remote_copy(
                src_ref=out_ref,
                dst_ref=ar_ref.at[s],
                send_sem=ar_send_sems.at[s],
                recv_sem=ar_recv_sems.at[s],
                device_id=(partner,),
                device_id_type=pl.DeviceIdType.MESH,
            )
            rdma.start()
            rdma.wait()
            out_ref[...] = out_ref[...] + ar_ref[s]

    return pl.pallas_call(
        body,
        out_shape=jax.ShapeDtypeStruct((B, Sq, Dm), F32),
        in_specs=[
            pl.BlockSpec(memory_space=pltpu.VMEM),
            pl.BlockSpec(memory_space=pltpu.VMEM),
            pl.BlockSpec(memory_space=pl.ANY),
            pl.BlockSpec(memory_space=pl.ANY),
            pl.BlockSpec(memory_space=pltpu.VMEM),
        ],
        out_specs=pl.BlockSpec(memory_space=pltpu.VMEM),
        scratch_shapes=[
            pltpu.VMEM((B, Sq, HL * DH), F32),
            pltpu.VMEM((N_DEV, B, Skv_l, HL, DH), F32),
            pltpu.VMEM((N_DEV, B, Skv_l, HL, DH), F32),
            pltpu.VMEM((B, Sq, HL, DH), F32),
            pltpu.VMEM((4, B, Sq, Dm), F32),
            pltpu.SemaphoreType.DMA((N_DEV,)),
            pltpu.SemaphoreType.DMA((N_DEV,)),
            pltpu.SemaphoreType.DMA((N_DEV,)),
            pltpu.SemaphoreType.DMA((N_DEV,)),
            pltpu.SemaphoreType.DMA((4,)),
            pltpu.SemaphoreType.DMA((4,)),
            pltpu.SemaphoreType.DMA((2,)),
        ],
        compiler_params=pltpu.CompilerParams(
            collective_id=0,
            vmem_limit_bytes=100 * 1024 * 1024,
        ),
    )(x, Wq, K_ext, V_ext, Wo)
